# speedup vs baseline: 1.5574x; 1.5574x over previous
import functools

import jax
import jax.numpy as jnp
from jax import lax
from jax.experimental import pallas as pl
from jax.experimental.pallas import tpu as pltpu
from jax.experimental.pallas import tpu_sc as plsc

_HIDDEN = 768
_B, _L = 4096, 20
_N = _B * _L

_NC, _NS, _LANES = 2, 16, 16
_NW = _NC * _NS
_PER_W = _N // _NW
_CHUNK = 64
_NCHUNK = _PER_W // _CHUNK

_mesh = plsc.VectorSubcoreMesh(core_axis_name="c", subcore_axis_name="s")


@functools.partial(
    pl.kernel,
    mesh=_mesh,
    out_type=jax.ShapeDtypeStruct((_N, _HIDDEN), jnp.float32),
    scratch_types=[
        pltpu.VMEM((_CHUNK,), jnp.int32),
        pltpu.VMEM((_CHUNK,), jnp.int32),
        pltpu.VMEM((_CHUNK, _HIDDEN), jnp.float32),
        pltpu.VMEM((_CHUNK, _HIDDEN), jnp.float32),
        pltpu.SemaphoreType.DMA,
        pltpu.SemaphoreType.DMA,
    ],
)
def _embed_sum(gsn_ids_hbm, route_ids_hbm, gsn_hbm, route_hbm, out_hbm,
               gidx_v, ridx_v, gbuf, rbuf, sem_g, sem_r):
    wid = lax.axis_index("s") * _NC + lax.axis_index("c")
    base = wid * _PER_W

    def chunk_body(ci, carry):
        off = base + ci * _CHUNK
        pltpu.sync_copy(gsn_ids_hbm.at[pl.ds(off, _CHUNK)], gidx_v)
        pltpu.sync_copy(route_ids_hbm.at[pl.ds(off, _CHUNK)], ridx_v)
        cg = pltpu.async_copy(gsn_hbm.at[gidx_v], gbuf, sem_g)
        cr = pltpu.async_copy(route_hbm.at[ridx_v], rbuf, sem_r)
        cg.wait()
        cr.wait()

        def row_body(i, c):
            for j in range(_HIDDEN // _LANES):
                sl = pl.ds(j * _LANES, _LANES)
                gbuf[i, sl] = gbuf[i, sl] + rbuf[i, sl]
            return c

        lax.fori_loop(0, _CHUNK, row_body, 0)
        pltpu.sync_copy(gbuf, out_hbm.at[pl.ds(off, _CHUNK)])
        return carry

    lax.fori_loop(0, _NCHUNK, chunk_body, 0)


def kernel(gsn_ids, route_ids, gsn_table, route_table):
    gi = gsn_ids.reshape(-1).astype(jnp.int32)
    ri = route_ids.reshape(-1).astype(jnp.int32)
    out = _embed_sum(gi, ri, gsn_table, route_table)
    return out.reshape(_B, _L, _HIDDEN)


# speedup vs baseline: 1.6937x; 1.0875x over previous
import functools

import jax
import jax.numpy as jnp
from jax import lax
from jax.experimental import pallas as pl
from jax.experimental.pallas import tpu as pltpu
from jax.experimental.pallas import tpu_sc as plsc

_HIDDEN = 768
_B, _L = 4096, 20
_N = _B * _L

_NC, _NS, _LANES = 2, 16, 16
_NW = _NC * _NS
_PER_W = _N // _NW
_CHUNK = 32
_NCHUNK = _PER_W // _CHUNK

_mesh = plsc.VectorSubcoreMesh(core_axis_name="c", subcore_axis_name="s")


@functools.partial(
    pl.kernel,
    mesh=_mesh,
    out_type=jax.ShapeDtypeStruct((_N, _HIDDEN), jnp.float32),
    scratch_types=[
        pltpu.VMEM((_NCHUNK, _CHUNK), jnp.int32),
        pltpu.VMEM((_NCHUNK, _CHUNK), jnp.int32),
        pltpu.VMEM((_CHUNK, _HIDDEN), jnp.float32),
        pltpu.VMEM((_CHUNK, _HIDDEN), jnp.float32),
        pltpu.VMEM((_CHUNK, _HIDDEN), jnp.float32),
        pltpu.VMEM((_CHUNK, _HIDDEN), jnp.float32),
        pltpu.SemaphoreType.DMA,
        pltpu.SemaphoreType.DMA,
        pltpu.SemaphoreType.DMA,
        pltpu.SemaphoreType.DMA,
        pltpu.SemaphoreType.DMA,
        pltpu.SemaphoreType.DMA,
    ],
)
def _embed_sum(gsn_ids_hbm, route_ids_hbm, gsn_hbm, route_hbm, out_hbm,
               gidx, ridx, gbuf0, rbuf0, gbuf1, rbuf1,
               sem_g0, sem_g1, sem_r0, sem_r1, sem_o0, sem_o1):
    wid = lax.axis_index("s") * _NC + lax.axis_index("c")
    base = wid * _PER_W

    pltpu.sync_copy(gsn_ids_hbm.at[wid], gidx)
    pltpu.sync_copy(route_ids_hbm.at[wid], ridx)

    def add_rows(gbuf, rbuf):
        def row_body(i, c):
            for j in range(_HIDDEN // _LANES):
                sl = pl.ds(j * _LANES, _LANES)
                gbuf[i, sl] = gbuf[i, sl] + rbuf[i, sl]
            return c
        lax.fori_loop(0, _CHUNK, row_body, 0)

    def group_body(g, carry):
        c0 = 2 * g
        c1 = c0 + 1
        dg0 = pltpu.async_copy(gsn_hbm.at[gidx.at[c0]], gbuf0, sem_g0)
        dr0 = pltpu.async_copy(route_hbm.at[ridx.at[c0]], rbuf0, sem_r0)
        dg1 = pltpu.async_copy(gsn_hbm.at[gidx.at[c1]], gbuf1, sem_g1)
        dr1 = pltpu.async_copy(route_hbm.at[ridx.at[c1]], rbuf1, sem_r1)

        dg0.wait()
        dr0.wait()
        add_rows(gbuf0, rbuf0)
        wb0 = pltpu.async_copy(
            gbuf0, out_hbm.at[pl.ds(base + c0 * _CHUNK, _CHUNK)], sem_o0)

        dg1.wait()
        dr1.wait()
        add_rows(gbuf1, rbuf1)
        wb1 = pltpu.async_copy(
            gbuf1, out_hbm.at[pl.ds(base + c1 * _CHUNK, _CHUNK)], sem_o1)

        wb0.wait()
        wb1.wait()
        return carry

    lax.fori_loop(0, _NCHUNK // 2, group_body, 0)


def kernel(gsn_ids, route_ids, gsn_table, route_table):
    gi = gsn_ids.reshape(_NW, _NCHUNK, _CHUNK).astype(jnp.int32)
    ri = route_ids.reshape(_NW, _NCHUNK, _CHUNK).astype(jnp.int32)
    out = _embed_sum(gi, ri, gsn_table, route_table)
    return out.reshape(_B, _L, _HIDDEN)
